# Initial kernel scaffold; baseline (speedup 1.0000x reference)
import jax
import jax.numpy as jnp
from jax.experimental import pallas as pl


def kernel(x_cont, x_cat, edge_index, batch, Wc1, bc1, Wc2, bc2, Wc3, bc3, Wk1, bk1, Wk2, bk2, Wk3, bk3, Wmc1, bmc1, Wmc2, bmc2, Wmk1, bmk1, Wmk2, bmk2, Wo1, bo1, Wo2, bo2, Wo3, bo3):
    raise NotImplementedError("write your pallas kernel here")


# speedup vs baseline: 3.4374x; 3.4374x over previous
import functools

import jax
import jax.numpy as jnp
from jax import lax
from jax.experimental import pallas as pl
from jax.experimental.pallas import tpu as pltpu
from jax.experimental.pallas import tpu_sc as plsc

N = 100000
E = 1600000
CONT = 128
CAT = 16
H = 32
W = 128

NC, NS, L = 2, 16, 16
NW = NC * NS
NB = 2000
EB = 2048
GCH = 128
EPW = 50048
E_PAD = EPW * NW
SCH = 2048
NR = 2 * NW
RPT = 1568
N_PAD = NR * RPT
NEG = float("-inf")

_mesh = plsc.VectorSubcoreMesh(
    core_axis_name="c", subcore_axis_name="s", num_cores=NC, num_subcores=NS)


def _node_mlp_body(xc, xk, Wc1, bc1, Wc2, bc2, Wc3, bc3,
                   Wk1, bk1, Wk2, bk2, Wk3, bk3, out):
    f32 = jnp.float32
    h = jax.nn.relu(jnp.dot(xc[...], Wc1[...], preferred_element_type=f32) + bc1[...])
    h = jax.nn.relu(jnp.dot(h, Wc2[...], preferred_element_type=f32) + bc2[...])
    hc = jnp.dot(h, Wc3[...], preferred_element_type=f32) + bc3[...]
    g = jax.nn.relu(jnp.dot(xk[...], Wk1[...], preferred_element_type=f32) + bk1[...])
    g = jax.nn.relu(jnp.dot(g, Wk2[...], preferred_element_type=f32) + bk2[...])
    hk = jnp.dot(g, Wk3[...], preferred_element_type=f32) + bk3[...]
    out[...] = jnp.concatenate(
        [hc, hk, jnp.zeros((hc.shape[0], W - 2 * H), f32)], axis=1)


def _full(shape):
    return pl.BlockSpec(shape, lambda i: (0, 0))


def _node_mlp(xc, xk, ws):
    specs = [pl.BlockSpec((NB, CONT), lambda i: (i, 0)),
             pl.BlockSpec((NB, CAT), lambda i: (i, 0))]
    specs += [_full(w.shape) for w in ws]
    return pl.pallas_call(
        _node_mlp_body,
        grid=(N // NB,),
        in_specs=specs,
        out_specs=pl.BlockSpec((NB, W), lambda i: (i, 0)),
        out_shape=jax.ShapeDtypeStruct((N, W), jnp.float32),
    )(xc, xk, *ws)


@functools.partial(
    pl.kernel,
    out_type=[jax.ShapeDtypeStruct((E_PAD, W), jnp.float32),
              jax.ShapeDtypeStruct((E_PAD, W), jnp.float32)],
    mesh=_mesh,
    scratch_types=[pltpu.VMEM((GCH,), jnp.int32),
                   pltpu.VMEM((GCH,), jnp.int32),
                   pltpu.VMEM((GCH, W), jnp.float32),
                   pltpu.VMEM((GCH, W), jnp.float32),
                   pltpu.SemaphoreType.DMA,
                   pltpu.SemaphoreType.DMA],
)
def _edge_gather(emb, dsti, srci, xi, xj, di_v, si_v, ri_v, rj_v, s1, s2):
    wid = lax.axis_index("s") * NC + lax.axis_index("c")
    base = wid * EPW

    def chunk(i, carry):
        off = base + i * GCH
        pltpu.sync_copy(dsti.at[pl.ds(off, GCH)], di_v)
        pltpu.sync_copy(srci.at[pl.ds(off, GCH)], si_v)
        c1 = pltpu.async_copy(emb.at[di_v], ri_v, s1)
        c2 = pltpu.async_copy(emb.at[si_v], rj_v, s2)
        c1.wait()
        c2.wait()
        pltpu.sync_copy(ri_v, xi.at[pl.ds(off, GCH)])
        pltpu.sync_copy(rj_v, xj.at[pl.ds(off, GCH)])
        return carry

    lax.fori_loop(0, EPW // GCH, chunk, 0)


def _edge_mlp_body(xi, xj, WA, WB, b1, W2, b2, m2):
    f32 = jnp.float32
    h = jax.nn.relu(jnp.dot(xi[...], WA[...], preferred_element_type=f32)
                    + jnp.dot(xj[...], WB[...], preferred_element_type=f32)
                    + b1[...])
    m2[...] = jnp.dot(h, W2[...], preferred_element_type=f32) + b2[...]


def _edge_mlp(xi, xj, ws):
    specs = [pl.BlockSpec((EB, W), lambda i: (i, 0))] * 2
    specs += [_full(w.shape) for w in ws]
    return pl.pallas_call(
        _edge_mlp_body,
        grid=(E_PAD // EB,),
        in_specs=specs,
        out_specs=pl.BlockSpec((EB, W), lambda i: (i, 0)),
        out_shape=jax.ShapeDtypeStruct((E_PAD, W), jnp.float32),
    )(xi, xj, *ws)


NBKT = 64
DCH = 6256
CAP = EPW + NBKT * 8
CAPP = CAP + GCH
PFW = 80
TRASH = RPT


@functools.partial(
    pl.kernel,
    out_type=[jax.ShapeDtypeStruct((NW * CAPP,), jnp.int32),
              jax.ShapeDtypeStruct((NW * CAPP,), jnp.int32),
              jax.ShapeDtypeStruct((NW * PFW,), jnp.int32)],
    mesh=_mesh,
    scratch_types=[pltpu.VMEM((DCH,), jnp.int32),
                   pltpu.VMEM((CAPP,), jnp.int32),
                   pltpu.VMEM((CAPP,), jnp.int32),
                   pltpu.VMEM((PFW * L,), jnp.int32),
                   pltpu.VMEM((PFW,), jnp.int32)],
)
def _bucket_sort(dsts, ids_o, dls_o, pfx_o, dbuf, idsb, dlsb, cbuf, pfxb):
    wid = lax.axis_index("s") * NC + lax.axis_index("c")
    base = wid * EPW
    zero16 = jnp.zeros((L,), dtype=jnp.int32)
    iota = lax.iota(jnp.int32, L)

    def ini(i, carry):
        cbuf[pl.ds(i * L, L)] = zero16
        return carry
    lax.fori_loop(0, PFW, ini, 0)

    def chunk1(ci, carry):
        pltpu.sync_copy(dsts.at[pl.ds(base + ci * DCH, DCH)], dbuf)

        def scan(v, carry2):
            dc = jnp.minimum(dbuf[pl.ds(v * L, L)], N)
            bv = ((dc >> 5) * 5350) >> 18
            for k in range(L):
                b16 = bv[k] * L
                cnt = cbuf[pl.ds(b16, L)][0]
                cbuf[pl.ds(b16, L)] = jnp.full((L,), cnt + 1, jnp.int32)
            return carry2

        lax.fori_loop(0, DCH // L, scan, 0)
        return carry
    lax.fori_loop(0, EPW // DCH, chunk1, 0)

    def pgrp(g, run):
        def lane(k, st):
            vec, r = st
            cnt = cbuf[pl.ds((g * L + k) * L, L)][0]
            vec = jnp.where(iota == k, r, vec)
            return vec, r + ((cnt + 7) // 8) * 8
        vec, run = lax.fori_loop(0, L, lane, (zero16, run))
        pfxb[pl.ds(g * L, L)] = vec
        return run
    total = lax.fori_loop(0, PFW // L, pgrp, 0)

    lax.fori_loop(0, PFW, ini, 0)

    def ini2(i, carry):
        idsb[pl.ds(i * L, L)] = zero16
        dlsb[pl.ds(i * L, L)] = zero16 + TRASH
        return carry
    lax.fori_loop(0, CAPP // L, ini2, 0)

    def chunk2(ci, carry):
        off = base + ci * DCH
        pltpu.sync_copy(dsts.at[pl.ds(off, DCH)], dbuf)

        def scan(v, carry2):
            dc = jnp.minimum(dbuf[pl.ds(v * L, L)], N)
            bv = ((dc >> 5) * 5350) >> 18
            dlv = dc - bv * RPT
            for k in range(L):
                b = bv[k]
                b16 = b * L
                cnt = cbuf[pl.ds(b16, L)][0]
                pos = pfxb[pl.ds(b, L)][0] + cnt
                idsb[pl.ds(pos, L)] = jnp.full(
                    (L,), off + v * L + k, jnp.int32)
                dlsb[pl.ds(pos, L)] = jnp.full((L,), dlv[k], jnp.int32)
                cbuf[pl.ds(b16, L)] = jnp.full((L,), cnt + 1, jnp.int32)
            return carry2

        lax.fori_loop(0, DCH // L, scan, 0)
        return carry
    lax.fori_loop(0, EPW // DCH, chunk2, 0)

    pltpu.sync_copy(idsb, ids_o.at[pl.ds(wid * CAPP, CAPP)])
    pltpu.sync_copy(dlsb, dls_o.at[pl.ds(wid * CAPP, CAPP)])
    pltpu.sync_copy(pfxb, pfx_o.at[pl.ds(wid * PFW, PFW)])


@functools.partial(
    pl.kernel,
    out_type=jax.ShapeDtypeStruct((N_PAD * 2 * H,), jnp.float32),
    mesh=_mesh,
    scratch_types=[pltpu.VMEM((NW * PFW + L,), jnp.int32),
                   pltpu.VMEM((GCH,), jnp.int32),
                   pltpu.VMEM((GCH + L,), jnp.int32),
                   pltpu.VMEM((GCH, W), jnp.float32),
                   pltpu.VMEM(((RPT + 8) * 2 * H,), jnp.float32),
                   pltpu.SemaphoreType.DMA],
)
def _segment_max(ids_i, dls_i, pfx_i, m2, agg, pfxv, gix, dlc, rows, acc, sem):
    wid = lax.axis_index("s") * NC + lax.axis_index("c")
    neg = jnp.full((L,), NEG, dtype=jnp.float32)
    AH = 2 * H
    pltpu.sync_copy(pfx_i, pfxv.at[pl.ds(0, NW * PFW)])

    for p in range(2):
        b = 2 * wid + p
        lo = b * RPT

        def ini(i, carry):
            acc[pl.ds(i * L, L)] = neg
            return carry
        lax.fori_loop(0, (RPT + 8) * AH // L, ini, 0)

        def producer(pt, carry):
            start = pfxv[pl.ds(pt * PFW + b, L)][0]
            end = pfxv[pl.ds(pt * PFW + b + 1, L)][0]
            rbase = pt * CAPP

            def gblk(g, carry2):
                gb = g * GCH

                @pl.when((gb + GCH > start) & (gb < end))
                def _():
                    pltpu.sync_copy(ids_i.at[pl.ds(rbase + gb, GCH)], gix)
                    pltpu.sync_copy(dls_i.at[pl.ds(rbase + gb, GCH)],
                                    dlc.at[pl.ds(0, GCH)])
                    pltpu.async_copy(m2.at[gix], rows, sem).wait()
                    e0 = jnp.maximum(start - gb, 0)
                    e1 = jnp.minimum(end - gb, GCH)

                    def rmw(e, carry4):
                        dl = dlc[pl.ds(e, L)][0]
                        for hh in range(AH // L):
                            r = rows[e, pl.ds(hh * L, L)]
                            a = acc[pl.ds(dl * AH + hh * L, L)]
                            acc[pl.ds(dl * AH + hh * L, L)] = jnp.maximum(a, r)
                        return carry4

                    lax.fori_loop(e0, e1, rmw, 0)
                return carry2

            lax.fori_loop(0, CAPP // GCH, gblk, 0)
            return carry

        lax.fori_loop(0, NW, producer, 0)
        pltpu.sync_copy(acc.at[pl.ds(0, RPT * AH)],
                        agg.at[pl.ds(lo * AH, RPT * AH)])


def _out_mlp_body(emb, ag, Wo1p, Wo1, bo1, Wo2, bo2, Wo3, bo3, out):
    f32 = jnp.float32
    a = ag[...]
    afix = jnp.where(a == NEG, jnp.float32(0), a)
    h = jax.nn.relu(jnp.dot(emb[...], Wo1p[...], preferred_element_type=f32)
                    + jnp.dot(afix, Wo1[...], preferred_element_type=f32)
                    + bo1[...])
    h = jax.nn.relu(jnp.dot(h, Wo2[...], preferred_element_type=f32) + bo2[...])
    out[...] = jnp.dot(h, Wo3[...], preferred_element_type=f32) + bo3[...]


def _out_mlp(emb, ag, ws):
    specs = [pl.BlockSpec((NB, W), lambda i: (i, 0)),
             pl.BlockSpec((NB, 2 * H), lambda i: (i, 0))]
    specs += [_full(w.shape) for w in ws]
    return pl.pallas_call(
        _out_mlp_body,
        grid=(N // NB,),
        in_specs=specs,
        out_specs=pl.BlockSpec((NB, 1), lambda i: (i, 0)),
        out_shape=jax.ShapeDtypeStruct((N, 1), jnp.float32),
    )(emb, ag, *ws)


def kernel(x_cont, x_cat, edge_index, batch,
           Wc1, bc1, Wc2, bc2, Wc3, bc3,
           Wk1, bk1, Wk2, bk2, Wk3, bk3,
           Wmc1, bmc1, Wmc2, bmc2,
           Wmk1, bmk1, Wmk2, bmk2,
           Wo1, bo1, Wo2, bo2, Wo3, bo3):
    f32 = jnp.float32
    row = lambda b: b.reshape(1, -1).astype(f32)

    emb = _node_mlp(x_cont, x_cat,
                    [Wc1, row(bc1), Wc2, row(bc2), Wc3, row(bc3),
                     Wk1, row(bk1), Wk2, row(bk2), Wk3, row(bk3)])

    src = edge_index[0]
    dst = edge_index[1]
    padg = jnp.zeros((E_PAD - E,), jnp.int32)
    dst_g = jnp.concatenate([dst, padg])
    src_g = jnp.concatenate([src, padg])
    dst_s = jnp.concatenate(
        [dst, jnp.full((E_PAD - E,), 2**31 - 1, jnp.int32)])

    xi, xj = _edge_gather(emb, dst_g, src_g)

    HH = 3 * H // 2
    zz = jnp.zeros((H, HH), f32)
    zp = jnp.zeros((W - 2 * H, 2 * HH), f32)
    WA = jnp.concatenate(
        [jnp.block([[Wmc1[:H] - Wmc1[H:], zz], [zz, Wmk1[:H] - Wmk1[H:]]]), zp],
        axis=0)
    WB = jnp.concatenate(
        [jnp.block([[Wmc1[H:], zz], [zz, Wmk1[H:]]]), zp], axis=0)
    b1 = jnp.concatenate([bmc1, bmk1]).reshape(1, -1)
    zh = jnp.zeros((HH, H), f32)
    W2 = jnp.concatenate(
        [jnp.concatenate([Wmc2, zh], axis=0),
         jnp.concatenate([zh, Wmk2], axis=0),
         jnp.zeros((2 * HH, W - 2 * H), f32)], axis=1)
    b2 = jnp.concatenate(
        [bmc2, bmk2, jnp.zeros((W - 2 * H,), f32)]).reshape(1, -1)

    m2 = _edge_mlp(xi, xj, [WA, WB, b1, W2, b2])

    eids, edls, epfx = _bucket_sort(dst_s)
    agg = _segment_max(eids, edls, epfx, m2).reshape(N_PAD, 2 * H)

    Wo1p = jnp.concatenate([Wo1, jnp.zeros((W - 2 * H, H), f32)], axis=0)
    out = _out_mlp(emb, agg,
                   [Wo1p, Wo1, row(bo1), Wo2, row(bo2), Wo3, row(bo3)])
    return out.reshape(N)


# speedup vs baseline: 3.6941x; 1.0747x over previous
import functools

import jax
import jax.numpy as jnp
from jax import lax
from jax.experimental import pallas as pl
from jax.experimental.pallas import tpu as pltpu
from jax.experimental.pallas import tpu_sc as plsc

N = 100000
E = 1600000
CONT = 128
CAT = 16
H = 32
W = 128
AH = 2 * H

NC, NS, L = 2, 16, 16
NW = NC * NS
NB = 2000
EB = 2048
GCH = 128
CPW = 392
EPW = CPW * GCH
E_PAD = EPW * NW
NCH = E_PAD // GCH
NBKT = 96
RPT = 1048
N_PAD = NBKT * RPT
DCH = 6272
CAP = EPW + NBKT * 8
CAPP = CAP + GCH
SWP = CAPP // GCH + 1
PFW = 104
TRASH = RPT
NEG = float("-inf")

_mesh = plsc.VectorSubcoreMesh(
    core_axis_name="c", subcore_axis_name="s", num_cores=NC, num_subcores=NS)


def _node_mlp_body(xc, xk, Wc1, bc1, Wc2, bc2, Wc3, bc3,
                   Wk1, bk1, Wk2, bk2, Wk3, bk3, out):
    f32 = jnp.float32
    h = jax.nn.relu(jnp.dot(xc[...], Wc1[...], preferred_element_type=f32) + bc1[...])
    h = jax.nn.relu(jnp.dot(h, Wc2[...], preferred_element_type=f32) + bc2[...])
    hc = jnp.dot(h, Wc3[...], preferred_element_type=f32) + bc3[...]
    g = jax.nn.relu(jnp.dot(xk[...], Wk1[...], preferred_element_type=f32) + bk1[...])
    g = jax.nn.relu(jnp.dot(g, Wk2[...], preferred_element_type=f32) + bk2[...])
    hk = jnp.dot(g, Wk3[...], preferred_element_type=f32) + bk3[...]
    out[...] = jnp.concatenate(
        [hc, hk, jnp.zeros((hc.shape[0], W - AH), f32)], axis=1)


def _full(shape):
    return pl.BlockSpec(shape, lambda i: (0, 0))


def _node_mlp(xc, xk, ws):
    specs = [pl.BlockSpec((NB, CONT), lambda i: (i, 0)),
             pl.BlockSpec((NB, CAT), lambda i: (i, 0))]
    specs += [_full(w.shape) for w in ws]
    return pl.pallas_call(
        _node_mlp_body,
        grid=(N // NB,),
        in_specs=specs,
        out_specs=pl.BlockSpec((NB, W), lambda i: (i, 0)),
        out_shape=jax.ShapeDtypeStruct((N, W), jnp.float32),
    )(xc, xk, *ws)


@functools.partial(
    pl.kernel,
    out_type=[jax.ShapeDtypeStruct((E_PAD, W), jnp.float32),
              jax.ShapeDtypeStruct((E_PAD, W), jnp.float32)],
    mesh=_mesh,
    scratch_types=[pltpu.VMEM((2, 2, GCH), jnp.int32),
                   pltpu.VMEM((GCH, W), jnp.float32),
                   pltpu.VMEM((GCH, W), jnp.float32),
                   pltpu.VMEM((GCH, W), jnp.float32),
                   pltpu.VMEM((GCH, W), jnp.float32),
                   pltpu.SemaphoreType.DMA,
                   pltpu.SemaphoreType.DMA,
                   pltpu.SemaphoreType.DMA,
                   pltpu.SemaphoreType.DMA,
                   pltpu.SemaphoreType.DMA,
                   pltpu.SemaphoreType.DMA,
                   pltpu.SemaphoreType.DMA,
                   pltpu.SemaphoreType.DMA,
                   pltpu.SemaphoreType.DMA,
                   pltpu.SemaphoreType.DMA],
)
def _edge_gather(emb, idx2, xi, xj, ixb, rd0, rs0, rd1, rs1,
                 si0, si1, sd0, ss0, sd1, ss1, wi0, wj0, wi1, wj1):
    wid = lax.axis_index("s") * NC + lax.axis_index("c")
    cbase = wid * CPW

    pltpu.async_copy(idx2.at[cbase], ixb.at[0], si0)
    pltpu.async_copy(idx2.at[cbase + 1], ixb.at[1], si1)

    def pair(gg, carry):
        g0 = 2 * gg
        for (g, par, sid, ssr, swi, swj, rd, rs, sem_i) in (
                (g0, 0, sd0, ss0, wi0, wj0, rd0, rs0, si0),
                (g0 + 1, 1, sd1, ss1, wi1, wj1, rd1, rs1, si1)):
            pltpu.make_async_copy(idx2.at[cbase + g], ixb.at[par], sem_i).wait()

            @pl.when(gg > 0)
            def _():
                pltpu.make_async_copy(rd, xi.at[pl.ds(0, GCH)], swi).wait()
                pltpu.make_async_copy(rs, xj.at[pl.ds(0, GCH)], swj).wait()

            pltpu.async_copy(emb.at[ixb.at[par, 0]], rd, sid)
            pltpu.async_copy(emb.at[ixb.at[par, 1]], rs, ssr)

        for (g, par, sid, ssr, swi, swj, rd, rs, sem_i) in (
                (g0, 0, sd0, ss0, wi0, wj0, rd0, rs0, si0),
                (g0 + 1, 1, sd1, ss1, wi1, wj1, rd1, rs1, si1)):
            off = (cbase + g) * GCH
            pltpu.make_async_copy(emb.at[pl.ds(0, GCH)], rd, sid).wait()
            pltpu.make_async_copy(emb.at[pl.ds(0, GCH)], rs, ssr).wait()
            pltpu.async_copy(rd, xi.at[pl.ds(off, GCH)], swi)
            pltpu.async_copy(rs, xj.at[pl.ds(off, GCH)], swj)

            @pl.when(gg < CPW // 2 - 1)
            def _():
                pltpu.async_copy(idx2.at[cbase + g + 2], ixb.at[par], sem_i)
        return carry

    lax.fori_loop(0, CPW // 2, pair, 0)
    for (swi, swj, rd, rs) in ((wi0, wj0, rd0, rs0), (wi1, wj1, rd1, rs1)):
        pltpu.make_async_copy(rd, xi.at[pl.ds(0, GCH)], swi).wait()
        pltpu.make_async_copy(rs, xj.at[pl.ds(0, GCH)], swj).wait()


def _edge_mlp_body(xi, xj, WA, WB, b1, W2, b2, m2):
    f32 = jnp.float32
    h = jax.nn.relu(jnp.dot(xi[...], WA[...], preferred_element_type=f32)
                    + jnp.dot(xj[...], WB[...], preferred_element_type=f32)
                    + b1[...])
    m2[...] = jnp.dot(h, W2[...], preferred_element_type=f32) + b2[...]


@functools.partial(
    pl.kernel,
    out_type=[jax.ShapeDtypeStruct((NW * CAPP,), jnp.int32),
              jax.ShapeDtypeStruct((NW * CAPP,), jnp.int32),
              jax.ShapeDtypeStruct((NW * PFW,), jnp.int32)],
    mesh=_mesh,
    scratch_types=[pltpu.VMEM((DCH,), jnp.int32),
                   pltpu.VMEM((CAPP,), jnp.int32),
                   pltpu.VMEM((CAPP,), jnp.int32),
                   pltpu.VMEM((PFW * L,), jnp.int32),
                   pltpu.VMEM((PFW,), jnp.int32)],
)
def _bucket_sort(dsts, ids_o, dls_o, pfx_o, dbuf, idsb, dlsb, cbuf, pfxb):
    wid = lax.axis_index("s") * NC + lax.axis_index("c")
    base = wid * EPW
    zero16 = jnp.zeros((L,), dtype=jnp.int32)
    iota = lax.iota(jnp.int32, L)

    def ini(i, carry):
        cbuf[pl.ds(i * L, L)] = zero16
        return carry
    lax.fori_loop(0, PFW, ini, 0)

    def chunk1(ci, carry):
        pltpu.sync_copy(dsts.at[pl.ds(base + ci * DCH, DCH)], dbuf)

        def scan(v, carry2):
            dc = jnp.minimum(dbuf[pl.ds(v * L, L)], N)
            bv = ((dc >> 3) * 128071) >> 24
            for k in range(L):
                b16 = bv[k] * L
                cnt = cbuf[pl.ds(b16, L)][0]
                cbuf[pl.ds(b16, L)] = jnp.full((L,), cnt + 1, jnp.int32)
            return carry2

        lax.fori_loop(0, DCH // L, scan, 0)
        return carry
    lax.fori_loop(0, EPW // DCH, chunk1, 0)

    def pgrp(g, run):
        def lane(k, st):
            vec, r = st
            cnt = cbuf[pl.ds((g * L + k) * L, L)][0]
            vec = jnp.where(iota == k, r, vec)
            return vec, r + ((cnt + 7) >> 3) * 8
        vec, run2 = lax.fori_loop(0, L, lane, (zero16, run))
        pfxb[pl.ds(g * L, L)] = vec
        return run2
    lax.fori_loop(0, PFW // L, pgrp, 0)

    lax.fori_loop(0, PFW, ini, 0)

    def ini2(i, carry):
        idsb[pl.ds(i * L, L)] = zero16
        dlsb[pl.ds(i * L, L)] = zero16 + TRASH
        return carry
    lax.fori_loop(0, CAPP // L, ini2, 0)

    def chunk2(ci, carry):
        off = base + ci * DCH
        pltpu.sync_copy(dsts.at[pl.ds(off, DCH)], dbuf)

        def scan(v, carry2):
            dc = jnp.minimum(dbuf[pl.ds(v * L, L)], N)
            bv = ((dc >> 3) * 128071) >> 24
            dlv = dc - bv * RPT
            for k in range(L):
                b = bv[k]
                b16 = b * L
                cnt = cbuf[pl.ds(b16, L)][0]
                pos = pfxb[pl.ds(b, L)][0] + cnt
                idsb[pl.ds(pos, L)] = jnp.full(
                    (L,), off + v * L + k, jnp.int32)
                dlsb[pl.ds(pos, L)] = jnp.full((L,), dlv[k], jnp.int32)
                cbuf[pl.ds(b16, L)] = jnp.full((L,), cnt + 1, jnp.int32)
            return carry2

        lax.fori_loop(0, DCH // L, scan, 0)
        return carry
    lax.fori_loop(0, EPW // DCH, chunk2, 0)

    pltpu.sync_copy(idsb, ids_o.at[pl.ds(wid * CAPP, CAPP)])
    pltpu.sync_copy(dlsb, dls_o.at[pl.ds(wid * CAPP, CAPP)])
    pltpu.sync_copy(pfxb, pfx_o.at[pl.ds(wid * PFW, PFW)])


@functools.partial(
    pl.kernel,
    out_type=jax.ShapeDtypeStruct((N_PAD * AH,), jnp.float32),
    mesh=_mesh,
    scratch_types=[pltpu.VMEM((PFW + L,), jnp.int32),
                   pltpu.VMEM((GCH,), jnp.int32),
                   pltpu.VMEM((GCH,), jnp.int32),
                   pltpu.VMEM((GCH + L,), jnp.int32),
                   pltpu.VMEM((GCH + L,), jnp.int32),
                   pltpu.VMEM((GCH, W), jnp.float32),
                   pltpu.VMEM((GCH, W), jnp.float32),
                   pltpu.VMEM(((RPT + 8) * AH,), jnp.float32),
                   pltpu.SemaphoreType.DMA,
                   pltpu.SemaphoreType.DMA,
                   pltpu.SemaphoreType.DMA,
                   pltpu.SemaphoreType.DMA,
                   pltpu.SemaphoreType.DMA,
                   pltpu.SemaphoreType.DMA],
)
def _segment_max(ids_i, dls_i, pfx_i, m2, agg, pfxb, gix0, gix1, dlc0, dlc1,
                 rows0, rows1, acc, sp0, sp1, sq0, sq1, sg0, sg1):
    wid = lax.axis_index("s") * NC + lax.axis_index("c")
    neg = jnp.full((L,), NEG, dtype=jnp.float32)

    for p in range(3):
        b = 3 * wid + p
        lo = b * RPT

        def ini(i, carry):
            acc[pl.ds(i * L, L)] = neg
            return carry
        lax.fori_loop(0, (RPT + 8) * AH // L, ini, 0)

        def producer(pt, carry):
            pltpu.sync_copy(pfx_i.at[pl.ds(pt * PFW, PFW)],
                            pfxb.at[pl.ds(0, PFW)])
            start = pfxb[pl.ds(b, L)][0]
            end = pfxb[pl.ds(b + 1, L)][0]
            rbase = pt * CAPP

            @pl.when((0 < end) & (GCH > start))
            def _():
                pltpu.async_copy(ids_i.at[pl.ds(rbase, GCH)], gix0, sp0)
                pltpu.async_copy(dls_i.at[pl.ds(rbase, GCH)],
                                 dlc0.at[pl.ds(0, GCH)], sq0)

            def pair(gg, carry2):
                g0 = 2 * gg
                for (g, gix, dlc, sp, sq, sg, srows, orows, so, ogix,
                     odlc, osp, osq) in (
                        (g0, gix0, dlc0, sp0, sq0, sg0, rows0, rows1, sg1,
                         gix1, dlc1, sp1, sq1),
                        (g0 + 1, gix1, dlc1, sp1, sq1, sg1, rows1, rows0,
                         sg0, gix0, dlc0, sp0, sq0)):
                    gb = g * GCH
                    inw = (gb < end) & (gb + GCH > start)
                    inw_n = (gb + GCH < end) & (gb + 2 * GCH > start)
                    inw_p = (gb - GCH < end) & (gb > start)

                    @pl.when(inw)
                    def _():
                        pltpu.make_async_copy(
                            ids_i.at[pl.ds(0, GCH)], gix, sp).wait()
                        pltpu.make_async_copy(
                            dls_i.at[pl.ds(0, GCH)],
                            dlc.at[pl.ds(0, GCH)], sq).wait()
                        pltpu.async_copy(m2.at[gix], srows, sg)

                    @pl.when(inw_p)
                    def _():
                        pltpu.make_async_copy(
                            m2.at[pl.ds(0, GCH)], orows, so).wait()
                        gbp = gb - GCH
                        e0 = jnp.maximum(start - gbp, 0)
                        e1 = jnp.minimum(end - gbp, GCH)

                        def rmw(e, c4):
                            dl = odlc[pl.ds(e, L)][0]
                            for hh in range(AH // L):
                                r = orows[e, pl.ds(hh * L, L)]
                                a = acc[pl.ds(dl * AH + hh * L, L)]
                                acc[pl.ds(dl * AH + hh * L, L)] = (
                                    jnp.maximum(a, r))
                            return c4

                        lax.fori_loop(e0, e1, rmw, 0)

                    @pl.when(inw_n)
                    def _():
                        pltpu.async_copy(
                            ids_i.at[pl.ds(rbase + gb + GCH, GCH)], ogix, osp)
                        pltpu.async_copy(
                            dls_i.at[pl.ds(rbase + gb + GCH, GCH)],
                            odlc.at[pl.ds(0, GCH)], osq)
                return carry2

            lax.fori_loop(0, SWP // 2, pair, 0)
            return carry

        lax.fori_loop(0, NW, producer, 0)
        pltpu.sync_copy(acc.at[pl.ds(0, RPT * AH)],
                        agg.at[pl.ds(lo * AH, RPT * AH)])


def _out_mlp_body(emb, ag, Wo1p, Wo1, bo1, Wo2, bo2, Wo3, bo3, out):
    f32 = jnp.float32
    a = ag[...]
    afix = jnp.where(a == NEG, jnp.float32(0), a)
    h = jax.nn.relu(jnp.dot(emb[...], Wo1p[...], preferred_element_type=f32)
                    + jnp.dot(afix, Wo1[...], preferred_element_type=f32)
                    + bo1[...])
    h = jax.nn.relu(jnp.dot(h, Wo2[...], preferred_element_type=f32) + bo2[...])
    out[...] = jnp.dot(h, Wo3[...], preferred_element_type=f32) + bo3[...]


def _out_mlp(emb, ag, ws):
    specs = [pl.BlockSpec((NB, W), lambda i: (i, 0)),
             pl.BlockSpec((NB, AH), lambda i: (i, 0))]
    specs += [_full(w.shape) for w in ws]
    return pl.pallas_call(
        _out_mlp_body,
        grid=(N // NB,),
        in_specs=specs,
        out_specs=pl.BlockSpec((NB, 1), lambda i: (i, 0)),
        out_shape=jax.ShapeDtypeStruct((N, 1), jnp.float32),
    )(emb, ag, *ws)


def kernel(x_cont, x_cat, edge_index, batch,
           Wc1, bc1, Wc2, bc2, Wc3, bc3,
           Wk1, bk1, Wk2, bk2, Wk3, bk3,
           Wmc1, bmc1, Wmc2, bmc2,
           Wmk1, bmk1, Wmk2, bmk2,
           Wo1, bo1, Wo2, bo2, Wo3, bo3):
    f32 = jnp.float32
    row = lambda b: b.reshape(1, -1).astype(f32)

    emb = _node_mlp(x_cont, x_cat,
                    [Wc1, row(bc1), Wc2, row(bc2), Wc3, row(bc3),
                     Wk1, row(bk1), Wk2, row(bk2), Wk3, row(bk3)])

    src = edge_index[0]
    dst = edge_index[1]
    padg = jnp.zeros((E_PAD - E,), jnp.int32)
    dst_g = jnp.concatenate([dst, padg])
    src_g = jnp.concatenate([src, padg])
    dst_s = jnp.concatenate(
        [dst, jnp.full((E_PAD - E,), 2**31 - 1, jnp.int32)])
    idx2 = jnp.stack(
        [dst_g.reshape(NCH, GCH), src_g.reshape(NCH, GCH)], axis=1)

    xi, xj = _edge_gather(emb, idx2)

    HH = 3 * H // 2
    zz = jnp.zeros((H, HH), f32)
    zp = jnp.zeros((W - AH, 2 * HH), f32)
    WA = jnp.concatenate(
        [jnp.block([[Wmc1[:H] - Wmc1[H:], zz], [zz, Wmk1[:H] - Wmk1[H:]]]), zp],
        axis=0)
    WB = jnp.concatenate(
        [jnp.block([[Wmc1[H:], zz], [zz, Wmk1[H:]]]), zp], axis=0)
    b1 = jnp.concatenate([bmc1, bmk1]).reshape(1, -1)
    zh = jnp.zeros((HH, H), f32)
    W2 = jnp.concatenate(
        [jnp.concatenate([Wmc2, zh], axis=0),
         jnp.concatenate([zh, Wmk2], axis=0),
         jnp.zeros((2 * HH, W - AH), f32)], axis=1)
    b2 = jnp.concatenate(
        [bmc2, bmk2, jnp.zeros((W - AH,), f32)]).reshape(1, -1)

    specs_ws = [WA, WB, b1, W2, b2]
    m2 = pl.pallas_call(
        _edge_mlp_body,
        grid=(E_PAD // EB,),
        in_specs=[pl.BlockSpec((EB, W), lambda i: (i, 0))] * 2
        + [_full(w.shape) for w in specs_ws],
        out_specs=pl.BlockSpec((EB, W), lambda i: (i, 0)),
        out_shape=jax.ShapeDtypeStruct((E_PAD, W), jnp.float32),
    )(xi, xj, *specs_ws)

    eids, edls, epfx = _bucket_sort(dst_s)
    agg = _segment_max(eids, edls, epfx, m2).reshape(N_PAD, AH)

    Wo1p = jnp.concatenate([Wo1, jnp.zeros((W - AH, H), f32)], axis=0)
    out = _out_mlp(emb, agg,
                   [Wo1p, Wo1, row(bo1), Wo2, row(bo2), Wo3, row(bo3)])
    return out.reshape(N)
